# baseline (speedup 1.0000x reference)
import functools

import jax
import jax.numpy as jnp
from jax import lax
from jax.experimental import pallas as pl
from jax.experimental.pallas import tpu as pltpu
from jax.experimental.pallas import tpu_sc as plsc

N = 10000
E = 320000
F_DIM = 128
KP = 16
NPAD = 10240
BLK = 256
NBLK = NPAD // BLK

NCORES = 2
NSUB = 16
NTILES = NCORES * NSUB
ECH = 128
EPT = 10240
RPT = NPAD // NSUB


@functools.lru_cache(None)
def _make_segsum(width, sweeps, mode):
  nch = EPT // ECH
  nsw = nch // sweeps
  nbuf = 1 if mode == "sync" else 2
  mesh = plsc.VectorSubcoreMesh(core_axis_name="c", subcore_axis_name="s",
                                num_cores=NCORES, num_subcores=NSUB)

  @functools.partial(
      pl.kernel,
      out_type=jax.ShapeDtypeStruct((NCORES, NPAD, width), jnp.float32),
      mesh=mesh,
      scratch_types=[
          pltpu.VMEM((nsw, ECH), jnp.int32),
          pltpu.VMEM((nsw, ECH), jnp.int32),
          [pltpu.VMEM((ECH, width), jnp.float32) for _ in range(nbuf)],
          pltpu.VMEM_SHARED((NPAD, width), jnp.float32),
          [pltpu.SemaphoreType.DMA for _ in range(2 * nbuf)],
      ],
      compiler_params=pltpu.CompilerParams(use_tc_tiling_on_sc=False),
  )
  def segsum(table, gidx, sidx, zeros, out, gv, sv, bufs, acc, sems):
    cid = lax.axis_index("c")
    sid = lax.axis_index("s")
    tix = cid * NSUB + sid
    pltpu.sync_copy(zeros.at[pl.ds(sid * RPT, RPT)],
                    acc.at[pl.ds(sid * RPT, RPT)])
    plsc.subcore_barrier()

    last = nsw - 1
    for sw in range(sweeps):
      pltpu.sync_copy(gidx.at[tix, pl.ds(sw * nsw, nsw)], gv)
      pltpu.sync_copy(sidx.at[tix, pl.ds(sw * nsw, nsw)], sv)

      if mode == "sync":
        buf, sg = bufs[0], sems[0]

        def body_sync(ci, carry):
          pltpu.async_copy(table.at[gv.at[ci]], buf, sg).wait()
          pltpu.sync_copy(buf, acc.at[sv.at[ci]], add=True)
          return carry

        lax.fori_loop(0, nsw, body_sync, 0)

      elif mode == "prefetch":
        buf0, buf1 = bufs
        s0, s1 = sems[0], sems[1]
        pltpu.async_copy(table.at[gv.at[0]], buf0, s0)
        pltpu.async_copy(table.at[gv.at[1]], buf1, s1)

        def body_pref(g, carry):
          i = 2 * g
          pltpu.make_async_copy(table.at[gv.at[i]], buf0, s0).wait()
          pltpu.sync_copy(buf0, acc.at[sv.at[i]], add=True)
          pltpu.async_copy(table.at[gv.at[jnp.minimum(i + 2, last)]], buf0, s0)
          pltpu.make_async_copy(table.at[gv.at[i + 1]], buf1, s1).wait()
          pltpu.sync_copy(buf1, acc.at[sv.at[i + 1]], add=True)
          pltpu.async_copy(table.at[gv.at[jnp.minimum(i + 3, last)]], buf1, s1)
          return carry

        lax.fori_loop(0, nsw // 2, body_pref, 0)
        pltpu.make_async_copy(table.at[gv.at[last]], buf0, s0).wait()
        pltpu.make_async_copy(table.at[gv.at[last]], buf1, s1).wait()

      else:
        buf0, buf1 = bufs
        g0, g1, c0, c1 = sems
        pltpu.async_copy(table.at[gv.at[0]], buf0, g0).wait()

        def body_asc(g, carry):
          i = 2 * g
          pltpu.async_copy(buf0, acc.at[sv.at[i]], c0, add=True)
          pltpu.async_copy(table.at[gv.at[jnp.minimum(i + 1, last)]],
                           buf1, g1).wait()
          pltpu.make_async_copy(buf0, acc.at[sv.at[i]], c0).wait()
          pltpu.async_copy(buf1, acc.at[sv.at[i + 1]], c1, add=True)
          pltpu.async_copy(table.at[gv.at[jnp.minimum(i + 2, last)]],
                           buf0, g0).wait()
          pltpu.make_async_copy(buf1, acc.at[sv.at[i + 1]], c1).wait()
          return carry

        lax.fori_loop(0, nsw // 2, body_asc, 0)

    plsc.subcore_barrier()
    pltpu.sync_copy(acc.at[pl.ds(sid * RPT, RPT)],
                    out.at[cid, pl.ds(sid * RPT, RPT)])

  return segsum


@functools.lru_cache(None)
def _make_segsum_super(width, sch):
  nch = EPT // ECH
  nsch = nch // sch
  mesh = plsc.VectorSubcoreMesh(core_axis_name="c", subcore_axis_name="s",
                                num_cores=NCORES, num_subcores=NSUB)

  @functools.partial(
      pl.kernel,
      out_type=jax.ShapeDtypeStruct((NCORES, NPAD, width), jnp.float32),
      mesh=mesh,
      scratch_types=[
          pltpu.VMEM((nsch, 1, sch * ECH), jnp.int32),
          pltpu.VMEM((nsch, 1, sch * ECH), jnp.int32),
          pltpu.VMEM((sch * ECH, width), jnp.float32),
          pltpu.VMEM((sch * ECH, width), jnp.float32),
          pltpu.VMEM_SHARED((NPAD, width), jnp.float32),
          pltpu.SemaphoreType.DMA,
          pltpu.SemaphoreType.DMA,
      ],
      compiler_params=pltpu.CompilerParams(use_tc_tiling_on_sc=False),
  )
  def segsum(table, gidx, sidx, zeros, out, gv, sv, buf0, buf1, acc, s0, s1):
    cid = lax.axis_index("c")
    sid = lax.axis_index("s")
    tix = cid * NSUB + sid
    pltpu.sync_copy(zeros.at[pl.ds(sid * RPT, RPT)],
                    acc.at[pl.ds(sid * RPT, RPT)])
    pltpu.sync_copy(gidx.at[tix], gv)
    pltpu.sync_copy(sidx.at[tix], sv)
    plsc.subcore_barrier()

    last = nsch - 1
    pltpu.async_copy(table.at[gv.at[0]], buf0, s0)
    pltpu.async_copy(table.at[gv.at[1]], buf1, s1)

    def body(g, carry):
      i = 2 * g
      pltpu.make_async_copy(table.at[gv.at[i]], buf0, s0).wait()
      pltpu.sync_copy(buf0, acc.at[sv.at[i]], add=True)
      pltpu.async_copy(table.at[gv.at[jnp.minimum(i + 2, last)]], buf0, s0)
      pltpu.make_async_copy(table.at[gv.at[i + 1]], buf1, s1).wait()
      pltpu.sync_copy(buf1, acc.at[sv.at[i + 1]], add=True)
      pltpu.async_copy(table.at[gv.at[jnp.minimum(i + 3, last)]], buf1, s1)
      return carry

    lax.fori_loop(0, nsch // 2, body, 0)
    pltpu.make_async_copy(table.at[gv.at[last]], buf0, s0).wait()
    pltpu.make_async_copy(table.at[gv.at[last]], buf1, s1).wait()
    plsc.subcore_barrier()
    pltpu.sync_copy(acc.at[pl.ds(sid * RPT, RPT)],
                    out.at[cid, pl.ds(sid * RPT, RPT)])

  return segsum


@functools.lru_cache(None)
def _make_degcount():
  mesh = plsc.VectorSubcoreMesh(core_axis_name="c", subcore_axis_name="s",
                                num_cores=NCORES, num_subcores=NSUB)

  @functools.partial(
      pl.kernel,
      out_type=jax.ShapeDtypeStruct((NCORES, NPAD, KP), jnp.float32),
      mesh=mesh,
      scratch_types=[
          pltpu.VMEM((EPT // ECH, ECH), jnp.int32),
          pltpu.VMEM((ECH, KP), jnp.float32),
          pltpu.VMEM_SHARED((NPAD, KP), jnp.float32),
      ],
      compiler_params=pltpu.CompilerParams(use_tc_tiling_on_sc=False),
  )
  def degcount(ones_hbm, sidx, zeros, out, sv, ones_v, acc):
    cid = lax.axis_index("c")
    sid = lax.axis_index("s")
    tix = cid * NSUB + sid
    pltpu.sync_copy(zeros.at[pl.ds(sid * RPT, RPT)],
                    acc.at[pl.ds(sid * RPT, RPT)])
    pltpu.sync_copy(sidx.at[tix], sv)
    pltpu.sync_copy(ones_hbm.at[pl.ds(0, ECH)], ones_v)
    plsc.subcore_barrier()

    def body(ci, carry):
      pltpu.sync_copy(ones_v, acc.at[sv.at[ci]], add=True)
      return carry

    lax.fori_loop(0, EPT // ECH, body, 0)
    plsc.subcore_barrier()
    pltpu.sync_copy(acc.at[pl.ds(sid * RPT, RPT)],
                    out.at[cid, pl.ds(sid * RPT, RPT)])

  return degcount


def _pad_edges(idx, ech=ECH):
  npad_e = NTILES * EPT - E
  pad_vals = N + (jnp.arange(npad_e, dtype=jnp.int32) % (NPAD - N))
  p = jnp.concatenate([idx, pad_vals])
  return p.reshape(NTILES, EPT // ech, ech)


def _dinv_from(d_ref):
  deg = d_ref[0, :, 0:1] + d_ref[1, :, 0:1] + 1.0
  return lax.rsqrt(deg)


def _tc_scale_xw(Xp, W1, degp):
  def body(x_ref, w_ref, d_ref, o_ref):
    dinv = _dinv_from(d_ref)
    xw = jnp.dot(x_ref[...], w_ref[...], preferred_element_type=jnp.float32)
    o_ref[...] = xw * dinv

  return pl.pallas_call(
      body,
      grid=(NBLK,),
      in_specs=[
          pl.BlockSpec((BLK, F_DIM), lambda i: (i, 0)),
          pl.BlockSpec((F_DIM, F_DIM), lambda i: (0, 0)),
          pl.BlockSpec((2, BLK, KP), lambda i: (0, i, 0)),
      ],
      out_specs=pl.BlockSpec((BLK, F_DIM), lambda i: (i, 0)),
      out_shape=jax.ShapeDtypeStruct((NPAD, F_DIM), jnp.float32),
  )(Xp, W1, degp)


def _tc_layer1_tail(aggp, y1, degp, b1r, W2p):
  def body(a_ref, y_ref, d_ref, b_ref, w_ref, o_ref):
    dinv = _dinv_from(d_ref)
    pre = dinv * (a_ref[0] + a_ref[1] + y_ref[...]) + b_ref[...]
    h = jnp.where(pre > 0, pre, jnp.exp(pre) - 1.0)
    o_ref[...] = jnp.dot(h, w_ref[...],
                         preferred_element_type=jnp.float32) * dinv

  return pl.pallas_call(
      body,
      grid=(NBLK,),
      in_specs=[
          pl.BlockSpec((2, BLK, F_DIM), lambda i: (0, i, 0)),
          pl.BlockSpec((BLK, F_DIM), lambda i: (i, 0)),
          pl.BlockSpec((2, BLK, KP), lambda i: (0, i, 0)),
          pl.BlockSpec((1, F_DIM), lambda i: (0, 0)),
          pl.BlockSpec((F_DIM, KP), lambda i: (0, 0)),
      ],
      out_specs=pl.BlockSpec((BLK, KP), lambda i: (i, 0)),
      out_shape=jax.ShapeDtypeStruct((NPAD, KP), jnp.float32),
  )(aggp, y1, degp, b1r, W2p)


def _tc_softmax_clusters(aggp, y2, degp, b2r, Mp):
  def body(a_ref, y_ref, d_ref, b_ref, m_ref, z_ref, xp_ref):
    dinv = _dinv_from(d_ref)
    zp = dinv * (a_ref[0] + a_ref[1] + y_ref[...]) + b_ref[...]
    mask = lax.broadcasted_iota(jnp.int32, (BLK, KP), 1) < 8
    zm = jnp.where(mask, zp, -1e30)
    zmax = jnp.max(zm, axis=1, keepdims=True)
    e = jnp.where(mask, jnp.exp(zp - zmax), 0.0)
    z = e / jnp.sum(e, axis=1, keepdims=True)
    z_ref[...] = z
    xp_ref[...] = jnp.dot(z, jnp.maximum(m_ref[...], 0.0),
                          preferred_element_type=jnp.float32)

  return pl.pallas_call(
      body,
      grid=(NBLK,),
      in_specs=[
          pl.BlockSpec((2, BLK, KP), lambda i: (0, i, 0)),
          pl.BlockSpec((BLK, KP), lambda i: (i, 0)),
          pl.BlockSpec((2, BLK, KP), lambda i: (0, i, 0)),
          pl.BlockSpec((1, KP), lambda i: (0, 0)),
          pl.BlockSpec((KP, KP), lambda i: (0, 0)),
      ],
      out_specs=[
          pl.BlockSpec((BLK, KP), lambda i: (i, 0)),
          pl.BlockSpec((BLK, KP), lambda i: (i, 0)),
      ],
      out_shape=[
          jax.ShapeDtypeStruct((NPAD, KP), jnp.float32),
          jax.ShapeDtypeStruct((NPAD, KP), jnp.float32),
      ],
  )(aggp, y2, degp, b2r, Mp)


def _tc_blend(Xpure, blurp, alpha11):
  def body(al_ref, x_ref, b_ref, o_ref):
    al = al_ref[0, 0]
    o_ref[...] = (1.0 - al) * x_ref[...] + al * (b_ref[0] + b_ref[1])

  return pl.pallas_call(
      body,
      grid=(NBLK,),
      in_specs=[
          pl.BlockSpec(memory_space=pltpu.SMEM),
          pl.BlockSpec((BLK, KP), lambda i: (i, 0)),
          pl.BlockSpec((2, BLK, KP), lambda i: (0, i, 0)),
      ],
      out_specs=pl.BlockSpec((BLK, KP), lambda i: (i, 0)),
      out_shape=jax.ShapeDtypeStruct((NPAD, KP), jnp.float32),
  )(alpha11, Xpure, blurp)


def kernel(X, ei_feat, ei_spatial, W1, b1, W2, b2, M, alpha):
  f32 = jnp.float32
  Xp = jnp.zeros((NPAD, F_DIM), f32).at[:N].set(X)
  W2p = jnp.zeros((F_DIM, KP), f32).at[:, :8].set(W2)
  b1r = b1.reshape(1, F_DIM)
  b2r = jnp.zeros((1, KP), f32).at[0, :8].set(b2)
  Mp = jnp.zeros((KP, KP), f32).at[:8, :8].set(M)
  alpha11 = jnp.asarray(alpha, f32).reshape(1, 1)

  rowf = _pad_edges(ei_feat[0])
  colf = _pad_edges(ei_feat[1])
  rows = _pad_edges(ei_spatial[0])
  cols = _pad_edges(ei_spatial[1])

  z16 = jnp.zeros((NPAD, KP), f32)
  z128 = jnp.zeros((NPAD, F_DIM), f32)
  ones16 = jnp.ones((NPAD, KP), f32)

  degp = _make_degcount()(ones16, colf, z16)
  y1 = _tc_scale_xw(Xp, W1, degp)
  agg1p = _make_segsum(F_DIM, 2, "ascat")(y1, rowf, colf, z128)
  y2 = _tc_layer1_tail(agg1p, y1, degp, b1r, W2p)
  agg2p = _make_segsum(KP, 1, "prefetch")(y2, rowf, colf, z16)
  Z, Xpure = _tc_softmax_clusters(agg2p, y2, degp, b2r, Mp)
  blurp = _make_segsum(KP, 1, "prefetch")(Xpure, cols, rows, z16)
  Xhat = _tc_blend(Xpure, blurp, alpha11)

  return (Z[:N, :8], Xhat[:N, :8])


# speedup vs baseline: 1.0494x; 1.0494x over previous
import functools

import jax
import jax.numpy as jnp
from jax import lax
from jax.experimental import pallas as pl
from jax.experimental.pallas import tpu as pltpu
from jax.experimental.pallas import tpu_sc as plsc

N = 10000
E = 320000
F_DIM = 128
KP = 16
NPAD = 10240
BLK = 256
NBLK = NPAD // BLK

NCORES = 2
NSUB = 16
NTILES = NCORES * NSUB
ECH = 128
EPT = 10240
RPT = NPAD // NSUB


@functools.lru_cache(None)
def _make_segsum(width, sweeps, mode):
  nch = EPT // ECH
  nsw = nch // sweeps
  nbuf = 1 if mode == "sync" else 2
  mesh = plsc.VectorSubcoreMesh(core_axis_name="c", subcore_axis_name="s",
                                num_cores=NCORES, num_subcores=NSUB)

  @functools.partial(
      pl.kernel,
      out_type=jax.ShapeDtypeStruct((NCORES, NPAD, width), jnp.float32),
      mesh=mesh,
      scratch_types=[
          pltpu.VMEM((nsw, ECH), jnp.int32),
          pltpu.VMEM((nsw, ECH), jnp.int32),
          [pltpu.VMEM((ECH, width), jnp.float32) for _ in range(nbuf)],
          pltpu.VMEM_SHARED((NPAD, width), jnp.float32),
          [pltpu.SemaphoreType.DMA for _ in range(2 * nbuf)],
      ],
      compiler_params=pltpu.CompilerParams(use_tc_tiling_on_sc=False),
  )
  def segsum(table, gidx, sidx, zeros, out, gv, sv, bufs, acc, sems):
    cid = lax.axis_index("c")
    sid = lax.axis_index("s")
    tix = cid * NSUB + sid
    pltpu.sync_copy(zeros.at[pl.ds(sid * RPT, RPT)],
                    acc.at[pl.ds(sid * RPT, RPT)])
    plsc.subcore_barrier()

    last = nsw - 1
    for sw in range(sweeps):
      pltpu.sync_copy(gidx.at[tix, pl.ds(sw * nsw, nsw)], gv)
      pltpu.sync_copy(sidx.at[tix, pl.ds(sw * nsw, nsw)], sv)

      if mode == "sync":
        buf, sg = bufs[0], sems[0]

        def body_sync(ci, carry):
          pltpu.async_copy(table.at[gv.at[ci]], buf, sg).wait()
          pltpu.sync_copy(buf, acc.at[sv.at[ci]], add=True)
          return carry

        lax.fori_loop(0, nsw, body_sync, 0)

      elif mode == "prefetch":
        buf0, buf1 = bufs
        s0, s1 = sems[0], sems[1]
        pltpu.async_copy(table.at[gv.at[0]], buf0, s0)
        pltpu.async_copy(table.at[gv.at[1]], buf1, s1)

        def body_pref(g, carry):
          i = 2 * g
          pltpu.make_async_copy(table.at[gv.at[i]], buf0, s0).wait()
          pltpu.sync_copy(buf0, acc.at[sv.at[i]], add=True)
          pltpu.async_copy(table.at[gv.at[jnp.minimum(i + 2, last)]], buf0, s0)
          pltpu.make_async_copy(table.at[gv.at[i + 1]], buf1, s1).wait()
          pltpu.sync_copy(buf1, acc.at[sv.at[i + 1]], add=True)
          pltpu.async_copy(table.at[gv.at[jnp.minimum(i + 3, last)]], buf1, s1)
          return carry

        lax.fori_loop(0, nsw // 2, body_pref, 0)
        pltpu.make_async_copy(table.at[gv.at[last]], buf0, s0).wait()
        pltpu.make_async_copy(table.at[gv.at[last]], buf1, s1).wait()

      else:
        buf0, buf1 = bufs
        g0, g1, c0, c1 = sems
        pltpu.async_copy(table.at[gv.at[0]], buf0, g0).wait()

        def body_asc(g, carry):
          i = 2 * g
          pltpu.async_copy(buf0, acc.at[sv.at[i]], c0, add=True)
          pltpu.async_copy(table.at[gv.at[jnp.minimum(i + 1, last)]],
                           buf1, g1).wait()
          pltpu.make_async_copy(buf0, acc.at[sv.at[i]], c0).wait()
          pltpu.async_copy(buf1, acc.at[sv.at[i + 1]], c1, add=True)
          pltpu.async_copy(table.at[gv.at[jnp.minimum(i + 2, last)]],
                           buf0, g0).wait()
          pltpu.make_async_copy(buf1, acc.at[sv.at[i + 1]], c1).wait()
          return carry

        lax.fori_loop(0, nsw // 2, body_asc, 0)

    plsc.subcore_barrier()
    pltpu.sync_copy(acc.at[pl.ds(sid * RPT, RPT)],
                    out.at[cid, pl.ds(sid * RPT, RPT)])

  return segsum


@functools.lru_cache(None)
def _make_degcount():
  mesh = plsc.VectorSubcoreMesh(core_axis_name="c", subcore_axis_name="s",
                                num_cores=NCORES, num_subcores=NSUB)

  @functools.partial(
      pl.kernel,
      out_type=jax.ShapeDtypeStruct((NCORES, NPAD, KP), jnp.float32),
      mesh=mesh,
      scratch_types=[
          pltpu.VMEM((EPT // ECH, ECH), jnp.int32),
          pltpu.VMEM((ECH, KP), jnp.float32),
          pltpu.VMEM_SHARED((NPAD, KP), jnp.float32),
      ],
      compiler_params=pltpu.CompilerParams(use_tc_tiling_on_sc=False),
  )
  def degcount(ones_hbm, sidx, zeros, out, sv, ones_v, acc):
    cid = lax.axis_index("c")
    sid = lax.axis_index("s")
    tix = cid * NSUB + sid
    pltpu.sync_copy(zeros.at[pl.ds(sid * RPT, RPT)],
                    acc.at[pl.ds(sid * RPT, RPT)])
    pltpu.sync_copy(sidx.at[tix], sv)
    pltpu.sync_copy(ones_hbm.at[pl.ds(0, ECH)], ones_v)
    plsc.subcore_barrier()

    def body(ci, carry):
      pltpu.sync_copy(ones_v, acc.at[sv.at[ci]], add=True)
      return carry

    lax.fori_loop(0, EPT // ECH, body, 0)
    plsc.subcore_barrier()
    pltpu.sync_copy(acc.at[pl.ds(sid * RPT, RPT)],
                    out.at[cid, pl.ds(sid * RPT, RPT)])

  return degcount


def _pad_edges(idx, ech=ECH):
  npad_e = NTILES * EPT - E
  pad_vals = N + (jnp.arange(npad_e, dtype=jnp.int32) % (NPAD - N))
  p = jnp.concatenate([idx, pad_vals])
  return p.reshape(NTILES, EPT // ech, ech)


def _dinv_from(d_ref):
  deg = d_ref[0, :, 0:1] + d_ref[1, :, 0:1] + 1.0
  return lax.rsqrt(deg)


def _tc_scale_xw(Xp, W1, degp):
  def body(x_ref, w_ref, d_ref, o_ref):
    dinv = _dinv_from(d_ref)
    xw = jnp.dot(x_ref[...], w_ref[...], preferred_element_type=jnp.float32)
    o_ref[...] = xw * dinv

  return pl.pallas_call(
      body,
      grid=(NBLK,),
      in_specs=[
          pl.BlockSpec((BLK, F_DIM), lambda i: (i, 0)),
          pl.BlockSpec((F_DIM, F_DIM), lambda i: (0, 0)),
          pl.BlockSpec((2, BLK, KP), lambda i: (0, i, 0)),
      ],
      out_specs=pl.BlockSpec((BLK, F_DIM), lambda i: (i, 0)),
      out_shape=jax.ShapeDtypeStruct((NPAD, F_DIM), jnp.float32),
  )(Xp, W1, degp)


def _tc_layer1_tail(aggp, y1, degp, b1r, W2p):
  def body(a_ref, y_ref, d_ref, b_ref, w_ref, o_ref):
    dinv = _dinv_from(d_ref)
    pre = dinv * (a_ref[0] + a_ref[1] + y_ref[...]) + b_ref[...]
    h = jnp.where(pre > 0, pre, jnp.exp(pre) - 1.0)
    o_ref[...] = jnp.dot(h, w_ref[...],
                         preferred_element_type=jnp.float32) * dinv

  return pl.pallas_call(
      body,
      grid=(NBLK,),
      in_specs=[
          pl.BlockSpec((2, BLK, F_DIM), lambda i: (0, i, 0)),
          pl.BlockSpec((BLK, F_DIM), lambda i: (i, 0)),
          pl.BlockSpec((2, BLK, KP), lambda i: (0, i, 0)),
          pl.BlockSpec((1, F_DIM), lambda i: (0, 0)),
          pl.BlockSpec((F_DIM, KP), lambda i: (0, 0)),
      ],
      out_specs=pl.BlockSpec((BLK, KP), lambda i: (i, 0)),
      out_shape=jax.ShapeDtypeStruct((NPAD, KP), jnp.float32),
  )(aggp, y1, degp, b1r, W2p)


def _tc_softmax_clusters(aggp, y2, degp, b2r, Mp):
  def body(a_ref, y_ref, d_ref, b_ref, m_ref, z_ref, xp_ref):
    dinv = _dinv_from(d_ref)
    zp = dinv * (a_ref[0] + a_ref[1] + y_ref[...]) + b_ref[...]
    mask = lax.broadcasted_iota(jnp.int32, (BLK, KP), 1) < 8
    zm = jnp.where(mask, zp, -1e30)
    zmax = jnp.max(zm, axis=1, keepdims=True)
    e = jnp.where(mask, jnp.exp(zp - zmax), 0.0)
    z = e / jnp.sum(e, axis=1, keepdims=True)
    z_ref[...] = z
    xp_ref[...] = jnp.dot(z, jnp.maximum(m_ref[...], 0.0),
                          preferred_element_type=jnp.float32)

  return pl.pallas_call(
      body,
      grid=(NBLK,),
      in_specs=[
          pl.BlockSpec((2, BLK, KP), lambda i: (0, i, 0)),
          pl.BlockSpec((BLK, KP), lambda i: (i, 0)),
          pl.BlockSpec((2, BLK, KP), lambda i: (0, i, 0)),
          pl.BlockSpec((1, KP), lambda i: (0, 0)),
          pl.BlockSpec((KP, KP), lambda i: (0, 0)),
      ],
      out_specs=[
          pl.BlockSpec((BLK, KP), lambda i: (i, 0)),
          pl.BlockSpec((BLK, KP), lambda i: (i, 0)),
      ],
      out_shape=[
          jax.ShapeDtypeStruct((NPAD, KP), jnp.float32),
          jax.ShapeDtypeStruct((NPAD, KP), jnp.float32),
      ],
  )(aggp, y2, degp, b2r, Mp)


def _tc_blend(Xpure, blurp, alpha11):
  def body(al_ref, x_ref, b_ref, o_ref):
    al = al_ref[0, 0]
    o_ref[...] = (1.0 - al) * x_ref[...] + al * (b_ref[0] + b_ref[1])

  return pl.pallas_call(
      body,
      grid=(NBLK,),
      in_specs=[
          pl.BlockSpec(memory_space=pltpu.SMEM),
          pl.BlockSpec((BLK, KP), lambda i: (i, 0)),
          pl.BlockSpec((2, BLK, KP), lambda i: (0, i, 0)),
      ],
      out_specs=pl.BlockSpec((BLK, KP), lambda i: (i, 0)),
      out_shape=jax.ShapeDtypeStruct((NPAD, KP), jnp.float32),
  )(alpha11, Xpure, blurp)


def kernel(X, ei_feat, ei_spatial, W1, b1, W2, b2, M, alpha):
  f32 = jnp.float32
  Xp = jnp.zeros((NPAD, F_DIM), f32).at[:N].set(X)
  W2p = jnp.zeros((F_DIM, KP), f32).at[:, :8].set(W2)
  b1r = b1.reshape(1, F_DIM)
  b2r = jnp.zeros((1, KP), f32).at[0, :8].set(b2)
  Mp = jnp.zeros((KP, KP), f32).at[:8, :8].set(M)
  alpha11 = jnp.asarray(alpha, f32).reshape(1, 1)

  rowf = _pad_edges(ei_feat[0])
  colf = _pad_edges(ei_feat[1])
  rows = _pad_edges(ei_spatial[0])
  cols = _pad_edges(ei_spatial[1])

  z16 = jnp.zeros((NPAD, KP), f32)
  z128 = jnp.zeros((NPAD, F_DIM), f32)
  ones16 = jnp.ones((NPAD, KP), f32)

  degp = _make_degcount()(ones16, colf, z16)
  y1 = _tc_scale_xw(Xp, W1, degp)
  agg1p = _make_segsum(F_DIM, 2, "prefetch")(y1, rowf, colf, z128)
  y2 = _tc_layer1_tail(agg1p, y1, degp, b1r, W2p)
  agg2p = _make_segsum(KP, 1, "prefetch")(y2, rowf, colf, z16)
  Z, Xpure = _tc_softmax_clusters(agg2p, y2, degp, b2r, Mp)
  blurp = _make_segsum(KP, 1, "prefetch")(Xpure, cols, rows, z16)
  Xhat = _tc_blend(Xpure, blurp, alpha11)

  return (Z[:N, :8], Xhat[:N, :8])
